# Initial kernel scaffold; baseline (speedup 1.0000x reference)
import jax
import jax.numpy as jnp
from jax.experimental import pallas as pl


def kernel(points, W, gamma, beta):
    raise NotImplementedError("write your pallas kernel here")


# speedup vs baseline: 1.1925x; 1.1925x over previous
import jax
import jax.numpy as jnp
from jax.experimental import pallas as pl

_B = 2
_GX = 512
_GY = 512
_NV = _GX * _GY
_D = 64
_TN = 8000

_VOXEL = 0.2
_PCMIN = -51.2


def _fwd_kernel(pts_ref, mean_ref, wt_ref, out_ref):
    pts = pts_ref[...]
    xy = pts[:, 1:3]
    coords = jnp.floor((xy - _PCMIN) / _VOXEL)
    centers = coords * _VOXEL + (_VOXEL / 2.0) + _PCMIN
    f_center = xy - centers
    f_cluster = pts[:, 1:4] - mean_ref[...]
    feats = jnp.concatenate([pts[:, 1:5], f_cluster, f_center], axis=1)
    out_ref[...] = jnp.dot(feats, wt_ref[...],
                           preferred_element_type=jnp.float32)


def _bn_relu_kernel(x_ref, a_ref, b_ref, o_ref):
    o_ref[...] = jnp.maximum(x_ref[...] * a_ref[...] + b_ref[...], 0.0)


def kernel(points, W, gamma, beta):
    n = points.shape[0]
    coords_f = (points[:, 1:3] - _PCMIN) / _VOXEL
    coords = coords_f.astype(jnp.int32)
    bidx = points[:, 0].astype(jnp.int32)
    pidx = bidx * _NV + coords[:, 1] * _GX + coords[:, 0]

    ones = jnp.ones((n,), dtype=jnp.float32)
    cnt = jax.ops.segment_sum(ones, pidx, num_segments=_B * _NV)
    sums = jax.ops.segment_sum(points[:, 1:4], pidx, num_segments=_B * _NV)
    mean = sums / jnp.maximum(cnt, 1.0)[:, None]
    mean_pts = mean[pidx]

    grid = (n // _TN,)
    x = pl.pallas_call(
        _fwd_kernel,
        grid=grid,
        in_specs=[
            pl.BlockSpec((_TN, 5), lambda i: (i, 0)),
            pl.BlockSpec((_TN, 3), lambda i: (i, 0)),
            pl.BlockSpec((9, _D), lambda i: (0, 0)),
        ],
        out_specs=pl.BlockSpec((_TN, _D), lambda i: (i, 0)),
        out_shape=jax.ShapeDtypeStruct((n, _D), jnp.float32),
    )(points, mean_pts, W.T)

    mu = jnp.mean(x, axis=0)
    var = jnp.mean((x - mu) ** 2, axis=0)
    a = gamma / jnp.sqrt(var + 1e-3)
    b = beta - mu * a

    y = pl.pallas_call(
        _bn_relu_kernel,
        grid=grid,
        in_specs=[
            pl.BlockSpec((_TN, _D), lambda i: (i, 0)),
            pl.BlockSpec((1, _D), lambda i: (0, 0)),
            pl.BlockSpec((1, _D), lambda i: (0, 0)),
        ],
        out_specs=pl.BlockSpec((_TN, _D), lambda i: (i, 0)),
        out_shape=jax.ShapeDtypeStruct((n, _D), jnp.float32),
    )(x, a[None, :], b[None, :])

    seg_max = jax.ops.segment_max(y, pidx, num_segments=_B * _NV)
    canvas = jnp.where(jnp.isfinite(seg_max), seg_max, 0.0)
    canvas = canvas.reshape(_B, _GY, _GX, _D).transpose(0, 3, 1, 2)
    return canvas
